# baseline (speedup 1.0000x reference)
import functools

import jax
import jax.numpy as jnp
from jax import lax
from jax.experimental import pallas as pl
from jax.experimental.pallas import tpu as pltpu
from jax.experimental.pallas import tpu_sc as plsc

T = 2048
C = 1024
E = 8
HID = 4096
NT = T * 2
BM = 256
NBUF = NT + E * BM
NB = NBUF // BM
NH = 4
HC = HID // NH
BT = 512
BS = 1024
_HI = jax.lax.Precision.HIGHEST


def _router_body(x_ref, wg_ref, pos_ref, w_ref, eob_ref, perm_ref):
    x = x_ref[...]
    wg = wg_ref[...]
    logits = lax.dot_general(x, wg, (((1,), (1,)), ((), ())))
    logits = jnp.clip(logits, -20.0, 20.0)
    e_iota = lax.broadcasted_iota(jnp.int32, (T, E), 1)
    m0 = jnp.max(logits, axis=1, keepdims=True)
    a0 = jnp.min(jnp.where(logits == m0, e_iota, E), axis=1, keepdims=True)
    masked = jnp.where(e_iota == a0, -jnp.inf, logits)
    m1 = jnp.max(masked, axis=1, keepdims=True)
    a1 = jnp.min(jnp.where(masked == m1, e_iota, E), axis=1, keepdims=True)
    r = jnp.exp(m1 - m0)
    w0 = 1.0 / (1.0 + r)
    w1 = r / (1.0 + r)
    w_ref[...] = jnp.concatenate([w0, w1], axis=1)

    ids = jnp.concatenate([a0, a1], axis=0)
    onehot = (ids == lax.broadcasted_iota(jnp.int32, (NT, E), 1)
              ).astype(jnp.float32)
    counts = jnp.sum(onehot, axis=0, keepdims=True)
    caps = jnp.ceil(counts / BM) * BM
    iu0 = lax.broadcasted_iota(jnp.int32, (E, E), 0)
    iu1 = lax.broadcasted_iota(jnp.int32, (E, E), 1)
    u_strict = (iu0 < iu1).astype(jnp.float32)
    u_incl = (iu0 <= iu1).astype(jnp.float32)
    offs = jnp.dot(caps, u_strict, precision=_HI)
    cumcaps = jnp.dot(caps, u_incl, precision=_HI)

    bio = (lax.broadcasted_iota(jnp.int32, (NB, E), 0) * BM).astype(jnp.float32)
    eob = jnp.sum((bio >= cumcaps).astype(jnp.int32), axis=1, keepdims=True)
    eob_ref[...] = jnp.minimum(eob, E - 1)

    CH = 512
    NCH = NT // CH
    pch = ((lax.broadcasted_iota(jnp.int32, (NCH, NT), 1) // CH)
           == lax.broadcasted_iota(jnp.int32, (NCH, NT), 0)
           ).astype(jnp.float32)
    s_all = jnp.dot(pch, onehot)
    l_tri = (lax.broadcasted_iota(jnp.int32, (NCH, NCH), 1)
             < lax.broadcasted_iota(jnp.int32, (NCH, NCH), 0)
             ).astype(jnp.float32)
    s_pre = jnp.dot(l_tri, s_all, precision=_HI)
    tri = (lax.broadcasted_iota(jnp.int32, (CH, CH), 1)
           < lax.broadcasted_iota(jnp.int32, (CH, CH), 0)
           ).astype(jnp.float32)
    pos_all = []
    for c in range(NCH):
        ohc = onehot[c * CH:(c + 1) * CH]
        rloc = jnp.dot(tri, ohc)
        base = s_pre[c:c + 1, :] + offs
        pos_c = jnp.sum((rloc + base) * ohc, axis=1, keepdims=True)
        pos_all.append(pos_c)
        pos_ref[pl.ds(c * CH, CH), :] = pos_c.astype(jnp.int32)

    pos_f = jnp.concatenate(pos_all, axis=0)
    tok = (lax.broadcasted_iota(jnp.int32, (NT, 1), 0) % T
           ).astype(jnp.float32)
    PCH = 512
    for c in range(NBUF // PCH):
        scol = (lax.broadcasted_iota(jnp.int32, (NT, PCH), 1)
                + c * PCH).astype(jnp.float32)
        ind = (pos_f == scol).astype(jnp.float32)
        perm_c = lax.dot_general(ind, tok, (((0,), (0,)), ((), ())),
                                 precision=_HI)
        perm_ref[pl.ds(c * PCH, PCH), :] = perm_c.astype(jnp.int32)


def _make_sc_gather():
    info = plsc.get_sparse_core_info()
    nc, ns = info.num_cores, info.num_subcores
    nw = nc * ns
    rows_per_w = NBUF // nw
    chunk = 64
    mesh = plsc.VectorSubcoreMesh(core_axis_name="c", subcore_axis_name="s")

    @functools.partial(
        pl.kernel, mesh=mesh,
        out_type=jax.ShapeDtypeStruct((NBUF, C), jnp.float32),
        scratch_types=[
            pltpu.VMEM((chunk,), jnp.int32),
            pltpu.VMEM((chunk, C), jnp.float32),
            pltpu.SemaphoreType.DMA,
        ],
    )
    def sc_gather(x_hbm, perm_hbm, xs_hbm, idx_v, rows_v, sem):
        wid = lax.axis_index("s") * nc + lax.axis_index("c")
        base = wid * rows_per_w
        for k in range(rows_per_w // chunk):
            off = base + k * chunk
            pltpu.sync_copy(perm_hbm.at[pl.ds(off, chunk)], idx_v)
            pltpu.async_copy(x_hbm.at[idx_v], rows_v, sem).wait()
            pltpu.sync_copy(rows_v, xs_hbm.at[pl.ds(off, chunk)])

    return sc_gather


_sc_gather = _make_sc_gather()


def _ffn_body(eob_ref, xs_ref, w1_ref, b1_ref, w2_ref, b2_ref,
              out_ref, acc_ref):
    h = pl.program_id(0)
    m = pl.program_id(1)
    hc = lax.dot_general(xs_ref[...], w1_ref[0],
                         (((1,), (1,)), ((), ()))) + b1_ref[0]
    hc = 0.5 * hc * (1.0 + lax.erf(hc * 0.7071067811865476))
    part = lax.dot_general(hc, w2_ref[0], (((1,), (1,)), ((), ())))

    @pl.when(h == 0)
    def _init():
        acc_ref[pl.ds(m * BM, BM), :] = part + b2_ref[0]

    @pl.when(h != 0)
    def _acc():
        acc_ref[pl.ds(m * BM, BM), :] += part

    @pl.when(h == NH - 1)
    def _flush():
        out_ref[...] = acc_ref[pl.ds(m * BM, BM), :]


def _combine_body(posT_ref, w_ref, outs_ref, res_ref):
    s = pl.program_id(1)
    p0 = posT_ref[:, 0:1]
    p1 = posT_ref[:, 1:2]
    w0 = w_ref[:, 0:1]
    w1 = w_ref[:, 1:2]
    scol = (lax.broadcasted_iota(jnp.int32, (p0.shape[0], BS), 1)
            + s * BS)
    cmask = ((scol == p0).astype(jnp.float32) * w0 +
             (scol == p1).astype(jnp.float32) * w1)
    part = jnp.dot(cmask, outs_ref[...])

    @pl.when(s == 0)
    def _init():
        res_ref[...] = part

    @pl.when(s != 0)
    def _acc():
        res_ref[...] += part


@jax.jit
def kernel(x, Wg, W1, b1, W2, b2):
    x_flat = x.reshape(T, C)
    pos, w, eob, perm = pl.pallas_call(
        _router_body,
        out_shape=[
            jax.ShapeDtypeStruct((NT, 1), jnp.int32),
            jax.ShapeDtypeStruct((T, 2), jnp.float32),
            jax.ShapeDtypeStruct((NB, 1), jnp.int32),
            jax.ShapeDtypeStruct((NBUF, 1), jnp.int32),
        ],
        compiler_params=pltpu.CompilerParams(
            vmem_limit_bytes=128 * 1024 * 1024),
    )(x_flat, Wg)

    pos2 = pos.reshape(2, T)
    posT = pos2.transpose(1, 0)
    eob1 = eob.reshape(NB)

    xs = _sc_gather(x_flat, perm.reshape(NBUF))

    outs = pl.pallas_call(
        _ffn_body,
        grid_spec=pltpu.PrefetchScalarGridSpec(
            num_scalar_prefetch=1,
            grid=(NH, NB),
            in_specs=[
                pl.BlockSpec((BM, C), lambda h, m, eob: (m, 0)),
                pl.BlockSpec((1, HC, C), lambda h, m, eob: (eob[m], h, 0)),
                pl.BlockSpec((1, 1, HC),
                             lambda h, m, eob: (eob[m] * NH + h, 0, 0)),
                pl.BlockSpec((1, C, HC), lambda h, m, eob: (eob[m], 0, h)),
                pl.BlockSpec((1, 1, C), lambda h, m, eob: (eob[m], 0, 0)),
            ],
            out_specs=pl.BlockSpec(
                (BM, C),
                lambda h, m, eob: (jnp.where(h == NH - 1, m, NB), 0)),
            scratch_shapes=[pltpu.VMEM((NBUF, C), jnp.float32)],
        ),
        out_shape=jax.ShapeDtypeStruct((NBUF + BM, C), jnp.float32),
        compiler_params=pltpu.CompilerParams(
            vmem_limit_bytes=128 * 1024 * 1024),
    )(eob1, xs, W1, b1.reshape(E * NH, 1, HC), W2,
      b2.reshape(E, 1, C))

    res = pl.pallas_call(
        _combine_body,
        grid=(T // BT, NBUF // BS),
        in_specs=[
            pl.BlockSpec((BT, 2), lambda i, s: (i, 0)),
            pl.BlockSpec((BT, 2), lambda i, s: (i, 0)),
            pl.BlockSpec((BS, C), lambda i, s: (s, 0)),
        ],
        out_specs=pl.BlockSpec((BT, C), lambda i, s: (i, 0)),
        out_shape=jax.ShapeDtypeStruct((T, C), jnp.float32),
        compiler_params=pltpu.CompilerParams(
            vmem_limit_bytes=128 * 1024 * 1024),
    )(posT, w, outs)
    return res.reshape(1, T, C)


# speedup vs baseline: 1.5375x; 1.5375x over previous
import functools

import jax
import jax.numpy as jnp
from jax import lax
from jax.experimental import pallas as pl
from jax.experimental.pallas import tpu as pltpu
from jax.experimental.pallas import tpu_sc as plsc

T = 2048
C = 1024
E = 8
HID = 4096
NT = T * 2
BM = 256
NBUF = NT + E * BM
NB = NBUF // BM
NH = 4
HC = HID // NH
BT = 512
BS = 1024
_HI = jax.lax.Precision.HIGHEST


def _router_body(x_ref, wg_ref, pos_ref, w_ref, eob_ref):
    x = x_ref[...]
    wg = wg_ref[...]
    logits = lax.dot_general(x, wg, (((1,), (1,)), ((), ())))
    logits = jnp.clip(logits, -20.0, 20.0)
    e_iota = lax.broadcasted_iota(jnp.int32, (T, E), 1)
    m0 = jnp.max(logits, axis=1, keepdims=True)
    a0 = jnp.min(jnp.where(logits == m0, e_iota, E), axis=1, keepdims=True)
    masked = jnp.where(e_iota == a0, -jnp.inf, logits)
    m1 = jnp.max(masked, axis=1, keepdims=True)
    a1 = jnp.min(jnp.where(masked == m1, e_iota, E), axis=1, keepdims=True)
    r = jnp.exp(m1 - m0)
    w0 = 1.0 / (1.0 + r)
    w1 = r / (1.0 + r)
    w_ref[...] = jnp.concatenate([w0, w1], axis=1)

    ids = jnp.concatenate([a0, a1], axis=0)
    onehot = (ids == lax.broadcasted_iota(jnp.int32, (NT, E), 1)
              ).astype(jnp.float32)
    counts = jnp.sum(onehot, axis=0, keepdims=True)
    caps = jnp.ceil(counts / BM) * BM
    iu0 = lax.broadcasted_iota(jnp.int32, (E, E), 0)
    iu1 = lax.broadcasted_iota(jnp.int32, (E, E), 1)
    u_strict = (iu0 < iu1).astype(jnp.float32)
    u_incl = (iu0 <= iu1).astype(jnp.float32)
    offs = jnp.dot(caps, u_strict, precision=_HI)
    cumcaps = jnp.dot(caps, u_incl, precision=_HI)

    bio = (lax.broadcasted_iota(jnp.int32, (NB, E), 0) * BM).astype(jnp.float32)
    eob = jnp.sum((bio >= cumcaps).astype(jnp.int32), axis=1, keepdims=True)
    eob_ref[...] = jnp.minimum(eob, E - 1)

    CH = 512
    NCH = NT // CH
    pch = ((lax.broadcasted_iota(jnp.int32, (NCH, NT), 1) // CH)
           == lax.broadcasted_iota(jnp.int32, (NCH, NT), 0)
           ).astype(jnp.float32)
    s_all = jnp.dot(pch, onehot)
    l_tri = (lax.broadcasted_iota(jnp.int32, (NCH, NCH), 1)
             < lax.broadcasted_iota(jnp.int32, (NCH, NCH), 0)
             ).astype(jnp.float32)
    s_pre = jnp.dot(l_tri, s_all, precision=_HI)
    tri = (lax.broadcasted_iota(jnp.int32, (CH, CH), 1)
           < lax.broadcasted_iota(jnp.int32, (CH, CH), 0)
           ).astype(jnp.float32)
    for c in range(NCH):
        ohc = onehot[c * CH:(c + 1) * CH]
        rloc = jnp.dot(tri, ohc)
        base = s_pre[c:c + 1, :] + offs
        pos_c = jnp.sum((rloc + base) * ohc, axis=1, keepdims=True)
        pos_ref[pl.ds(c * CH, CH), :] = pos_c.astype(jnp.int32)


def _dispatch_body(pos_ref, x_ref, xs_ref):
    m = pl.program_id(0)
    s0 = m * BM
    pos0 = pos_ref[0:1, :]
    pos1 = pos_ref[1:2, :]
    srow = lax.broadcasted_iota(jnp.int32, (BM, T), 0) + s0
    dmask = ((pos0 == srow) | (pos1 == srow)).astype(jnp.float32)
    xs_ref[...] = jnp.dot(dmask, x_ref[...])


def _ffn_body(eob_ref, xs_ref, w1_ref, b1_ref, w2_ref, b2_ref,
              out_ref, acc_ref):
    h = pl.program_id(0)
    m = pl.program_id(1)
    hc = lax.dot_general(xs_ref[...], w1_ref[0],
                         (((1,), (1,)), ((), ()))) + b1_ref[0]
    hc = 0.5 * hc * (1.0 + lax.erf(hc * 0.7071067811865476))
    part = lax.dot_general(hc, w2_ref[0], (((1,), (1,)), ((), ())))

    @pl.when(h == 0)
    def _init():
        acc_ref[pl.ds(m * BM, BM), :] = part + b2_ref[0]

    @pl.when(h != 0)
    def _acc():
        acc_ref[pl.ds(m * BM, BM), :] += part

    @pl.when(h == NH - 1)
    def _flush():
        out_ref[...] = acc_ref[pl.ds(m * BM, BM), :]


def _combine_body(posT_ref, w_ref, outs_ref, res_ref):
    s = pl.program_id(1)
    p0 = posT_ref[:, 0:1]
    p1 = posT_ref[:, 1:2]
    w0 = w_ref[:, 0:1]
    w1 = w_ref[:, 1:2]
    scol = (lax.broadcasted_iota(jnp.int32, (p0.shape[0], BS), 1)
            + s * BS)
    cmask = ((scol == p0).astype(jnp.float32) * w0 +
             (scol == p1).astype(jnp.float32) * w1)
    part = jnp.dot(cmask, outs_ref[...])

    @pl.when(s == 0)
    def _init():
        res_ref[...] = part

    @pl.when(s != 0)
    def _acc():
        res_ref[...] += part


@jax.jit
def kernel(x, Wg, W1, b1, W2, b2):
    x_flat = x.reshape(T, C)
    pos, w, eob = pl.pallas_call(
        _router_body,
        out_shape=[
            jax.ShapeDtypeStruct((NT, 1), jnp.int32),
            jax.ShapeDtypeStruct((T, 2), jnp.float32),
            jax.ShapeDtypeStruct((NB, 1), jnp.int32),
        ],
        compiler_params=pltpu.CompilerParams(
            vmem_limit_bytes=128 * 1024 * 1024),
    )(x_flat, Wg)

    pos2 = pos.reshape(2, T)
    posT = pos2.transpose(1, 0)
    eob1 = eob.reshape(NB)

    xs = pl.pallas_call(
        _dispatch_body,
        grid=(NB,),
        in_specs=[
            pl.BlockSpec((2, T), lambda m: (0, 0)),
            pl.BlockSpec((T, C), lambda m: (0, 0)),
        ],
        out_specs=pl.BlockSpec((BM, C), lambda m: (m, 0)),
        out_shape=jax.ShapeDtypeStruct((NBUF, C), jnp.float32),
        compiler_params=pltpu.CompilerParams(
            vmem_limit_bytes=128 * 1024 * 1024),
    )(pos2, x_flat)

    outs = pl.pallas_call(
        _ffn_body,
        grid_spec=pltpu.PrefetchScalarGridSpec(
            num_scalar_prefetch=1,
            grid=(NH, NB),
            in_specs=[
                pl.BlockSpec((BM, C), lambda h, m, eob: (m, 0)),
                pl.BlockSpec((1, HC, C), lambda h, m, eob: (eob[m], h, 0)),
                pl.BlockSpec((1, 1, HC),
                             lambda h, m, eob: (eob[m] * NH + h, 0, 0)),
                pl.BlockSpec((1, C, HC), lambda h, m, eob: (eob[m], 0, h)),
                pl.BlockSpec((1, 1, C), lambda h, m, eob: (eob[m], 0, 0)),
            ],
            out_specs=pl.BlockSpec(
                (BM, C),
                lambda h, m, eob: (jnp.where(h == NH - 1, m, NB), 0)),
            scratch_shapes=[pltpu.VMEM((NBUF, C), jnp.float32)],
        ),
        out_shape=jax.ShapeDtypeStruct((NBUF + BM, C), jnp.float32),
        compiler_params=pltpu.CompilerParams(
            vmem_limit_bytes=128 * 1024 * 1024),
    )(eob1, xs, W1, b1.reshape(E * NH, 1, HC), W2,
      b2.reshape(E, 1, C))

    res = pl.pallas_call(
        _combine_body,
        grid=(T // BT, NBUF // BS),
        in_specs=[
            pl.BlockSpec((BT, 2), lambda i, s: (i, 0)),
            pl.BlockSpec((BT, 2), lambda i, s: (i, 0)),
            pl.BlockSpec((BS, C), lambda i, s: (s, 0)),
        ],
        out_specs=pl.BlockSpec((BT, C), lambda i, s: (i, 0)),
        out_shape=jax.ShapeDtypeStruct((T, C), jnp.float32),
        compiler_params=pltpu.CompilerParams(
            vmem_limit_bytes=128 * 1024 * 1024),
    )(posT, w, outs)
    return res.reshape(1, T, C)


# speedup vs baseline: 1.6091x; 1.0466x over previous
import functools

import jax
import jax.numpy as jnp
from jax import lax
from jax.experimental import pallas as pl
from jax.experimental.pallas import tpu as pltpu
from jax.experimental.pallas import tpu_sc as plsc

T = 2048
C = 1024
E = 8
HID = 4096
NT = T * 2
BM = 256
NBUF = NT + E * BM
NB = NBUF // BM
NH = 4
HC = HID // NH
BT = 512
BS = 1024
_HI = jax.lax.Precision.HIGHEST


def _router_body(x_ref, wg_ref, pos_ref, w_ref, eob_ref, act_ref):
    x = x_ref[...]
    wg = wg_ref[...]
    logits = lax.dot_general(x, wg, (((1,), (1,)), ((), ())))
    logits = jnp.clip(logits, -20.0, 20.0)
    e_iota = lax.broadcasted_iota(jnp.int32, (T, E), 1)
    m0 = jnp.max(logits, axis=1, keepdims=True)
    a0 = jnp.min(jnp.where(logits == m0, e_iota, E), axis=1, keepdims=True)
    masked = jnp.where(e_iota == a0, -jnp.inf, logits)
    m1 = jnp.max(masked, axis=1, keepdims=True)
    a1 = jnp.min(jnp.where(masked == m1, e_iota, E), axis=1, keepdims=True)
    r = jnp.exp(m1 - m0)
    w0 = 1.0 / (1.0 + r)
    w1 = r / (1.0 + r)
    w_ref[...] = jnp.concatenate([w0, w1], axis=1)

    ids = jnp.concatenate([a0, a1], axis=0)
    onehot = (ids == lax.broadcasted_iota(jnp.int32, (NT, E), 1)
              ).astype(jnp.float32)
    counts = jnp.sum(onehot, axis=0, keepdims=True)
    caps = jnp.ceil(counts / BM) * BM
    iu0 = lax.broadcasted_iota(jnp.int32, (E, E), 0)
    iu1 = lax.broadcasted_iota(jnp.int32, (E, E), 1)
    u_strict = (iu0 < iu1).astype(jnp.float32)
    u_incl = (iu0 <= iu1).astype(jnp.float32)
    offs = jnp.dot(caps, u_strict, precision=_HI)
    cumcaps = jnp.dot(caps, u_incl, precision=_HI)

    bio = (lax.broadcasted_iota(jnp.int32, (NB, E), 0) * BM).astype(jnp.float32)
    eob = jnp.sum((bio >= cumcaps).astype(jnp.int32), axis=1, keepdims=True)
    eob_ref[...] = jnp.minimum(eob, E - 1)
    act_ref[...] = (bio[:, 0:1] < cumcaps[:, E - 1:E]).astype(jnp.int32)

    CH = 512
    NCH = NT // CH
    pch = ((lax.broadcasted_iota(jnp.int32, (NCH, NT), 1) // CH)
           == lax.broadcasted_iota(jnp.int32, (NCH, NT), 0)
           ).astype(jnp.float32)
    s_all = jnp.dot(pch, onehot)
    l_tri = (lax.broadcasted_iota(jnp.int32, (NCH, NCH), 1)
             < lax.broadcasted_iota(jnp.int32, (NCH, NCH), 0)
             ).astype(jnp.float32)
    s_pre = jnp.dot(l_tri, s_all, precision=_HI)
    tri = (lax.broadcasted_iota(jnp.int32, (CH, CH), 1)
           < lax.broadcasted_iota(jnp.int32, (CH, CH), 0)
           ).astype(jnp.float32)
    for c in range(NCH):
        ohc = onehot[c * CH:(c + 1) * CH]
        rloc = jnp.dot(tri, ohc)
        base = s_pre[c:c + 1, :] + offs
        pos_c = jnp.sum((rloc + base) * ohc, axis=1, keepdims=True)
        pos_ref[pl.ds(c * CH, CH), :] = pos_c.astype(jnp.int32)


def _dispatch_body(act_ref, pos_ref, x_ref, xs_ref):
    m = pl.program_id(0)

    @pl.when(act_ref[m] == 1)
    def _():
        s0 = m * BM
        pos0 = pos_ref[0:1, :]
        pos1 = pos_ref[1:2, :]
        srow = lax.broadcasted_iota(jnp.int32, (BM, T), 0) + s0
        dmask = ((pos0 == srow) | (pos1 == srow)).astype(jnp.float32)
        xs_ref[...] = jnp.dot(dmask, x_ref[...])

    @pl.when(act_ref[m] == 0)
    def _z():
        xs_ref[...] = jnp.zeros((BM, C), jnp.float32)


def _ffn_body(eob_ref, act_ref, xs_ref, w1_ref, b1_ref, w2_ref, b2_ref,
              out_ref, acc_ref):
    h = pl.program_id(0)
    m = pl.program_id(1)

    @pl.when(act_ref[m] == 1)
    def _compute():
        hc = lax.dot_general(xs_ref[...], w1_ref[0],
                             (((1,), (1,)), ((), ()))) + b1_ref[0]
        hc = 0.5 * hc * (1.0 + lax.erf(hc * 0.7071067811865476))
        part = lax.dot_general(hc, w2_ref[0], (((1,), (1,)), ((), ())))

        @pl.when(h == 0)
        def _init():
            acc_ref[pl.ds(m * BM, BM), :] = part + b2_ref[0]

        @pl.when(h != 0)
        def _acc():
            acc_ref[pl.ds(m * BM, BM), :] += part

        @pl.when(h == NH - 1)
        def _flush():
            out_ref[...] = acc_ref[pl.ds(m * BM, BM), :]

    @pl.when((act_ref[m] == 0) & (h == NH - 1))
    def _zero():
        out_ref[...] = jnp.zeros((BM, C), jnp.float32)


def _combine_body(posT_ref, w_ref, outs_ref, res_ref):
    s = pl.program_id(1)
    p0 = posT_ref[:, 0:1]
    p1 = posT_ref[:, 1:2]
    w0 = w_ref[:, 0:1]
    w1 = w_ref[:, 1:2]
    scol = (lax.broadcasted_iota(jnp.int32, (p0.shape[0], BS), 1)
            + s * BS)
    cmask = ((scol == p0).astype(jnp.float32) * w0 +
             (scol == p1).astype(jnp.float32) * w1)
    part = jnp.dot(cmask, outs_ref[...])

    @pl.when(s == 0)
    def _init():
        res_ref[...] = part

    @pl.when(s != 0)
    def _acc():
        res_ref[...] += part


@jax.jit
def kernel(x, Wg, W1, b1, W2, b2):
    x_flat = x.reshape(T, C)
    pos, w, eob, act = pl.pallas_call(
        _router_body,
        out_shape=[
            jax.ShapeDtypeStruct((NT, 1), jnp.int32),
            jax.ShapeDtypeStruct((T, 2), jnp.float32),
            jax.ShapeDtypeStruct((NB, 1), jnp.int32),
            jax.ShapeDtypeStruct((NB, 1), jnp.int32),
        ],
        compiler_params=pltpu.CompilerParams(
            vmem_limit_bytes=128 * 1024 * 1024),
    )(x_flat, Wg)

    pos2 = pos.reshape(2, T)
    posT = pos2.transpose(1, 0)
    eob1 = eob.reshape(NB)
    act1 = act.reshape(NB)

    xs = pl.pallas_call(
        _dispatch_body,
        grid_spec=pltpu.PrefetchScalarGridSpec(
            num_scalar_prefetch=1,
            grid=(NB,),
            in_specs=[
                pl.BlockSpec((2, T), lambda m, act: (0, 0)),
                pl.BlockSpec((T, C), lambda m, act: (0, 0)),
            ],
            out_specs=pl.BlockSpec((BM, C), lambda m, act: (m, 0)),
        ),
        out_shape=jax.ShapeDtypeStruct((NBUF, C), jnp.float32),
        compiler_params=pltpu.CompilerParams(
            vmem_limit_bytes=128 * 1024 * 1024),
    )(act1, pos2, x_flat)

    outs = pl.pallas_call(
        _ffn_body,
        grid_spec=pltpu.PrefetchScalarGridSpec(
            num_scalar_prefetch=2,
            grid=(NH, NB),
            in_specs=[
                pl.BlockSpec((BM, C), lambda h, m, eob, act: (m, 0)),
                pl.BlockSpec((1, HC, C),
                             lambda h, m, eob, act: (eob[m], h, 0)),
                pl.BlockSpec((1, 1, HC),
                             lambda h, m, eob, act: (eob[m] * NH + h, 0, 0)),
                pl.BlockSpec((1, C, HC),
                             lambda h, m, eob, act: (eob[m], 0, h)),
                pl.BlockSpec((1, 1, C),
                             lambda h, m, eob, act: (eob[m], 0, 0)),
            ],
            out_specs=pl.BlockSpec(
                (BM, C),
                lambda h, m, eob, act: (jnp.where(h == NH - 1, m, NB), 0)),
            scratch_shapes=[pltpu.VMEM((NBUF, C), jnp.float32)],
        ),
        out_shape=jax.ShapeDtypeStruct((NBUF + BM, C), jnp.float32),
        compiler_params=pltpu.CompilerParams(
            vmem_limit_bytes=128 * 1024 * 1024),
    )(eob1, act1, xs, W1, b1.reshape(E * NH, 1, HC), W2,
      b2.reshape(E, 1, C))

    res = pl.pallas_call(
        _combine_body,
        grid=(T // BT, NBUF // BS),
        in_specs=[
            pl.BlockSpec((BT, 2), lambda i, s: (i, 0)),
            pl.BlockSpec((BT, 2), lambda i, s: (i, 0)),
            pl.BlockSpec((BS, C), lambda i, s: (s, 0)),
        ],
        out_specs=pl.BlockSpec((BT, C), lambda i, s: (i, 0)),
        out_shape=jax.ShapeDtypeStruct((T, C), jnp.float32),
        compiler_params=pltpu.CompilerParams(
            vmem_limit_bytes=128 * 1024 * 1024),
    )(posT, w, outs)
    return res.reshape(1, T, C)


# speedup vs baseline: 1.6546x; 1.0282x over previous
import functools

import jax
import jax.numpy as jnp
from jax import lax
from jax.experimental import pallas as pl
from jax.experimental.pallas import tpu as pltpu
from jax.experimental.pallas import tpu_sc as plsc

T = 2048
C = 1024
E = 8
HID = 4096
NT = T * 2
BM = 256
NBUF = NT + E * BM
NB = NBUF // BM
NH = 4
HC = HID // NH
BT = 1024
BS = 1024
_HI = jax.lax.Precision.HIGHEST


def _router_body(x_ref, wg_ref, pos_ref, w_ref, eob_ref, act_ref):
    x = x_ref[...]
    wg = wg_ref[...]
    logits = lax.dot_general(x, wg, (((1,), (1,)), ((), ())))
    logits = jnp.clip(logits, -20.0, 20.0)
    e_iota = lax.broadcasted_iota(jnp.int32, (T, E), 1)
    m0 = jnp.max(logits, axis=1, keepdims=True)
    a0 = jnp.min(jnp.where(logits == m0, e_iota, E), axis=1, keepdims=True)
    masked = jnp.where(e_iota == a0, -jnp.inf, logits)
    m1 = jnp.max(masked, axis=1, keepdims=True)
    a1 = jnp.min(jnp.where(masked == m1, e_iota, E), axis=1, keepdims=True)
    r = jnp.exp(m1 - m0)
    w0 = 1.0 / (1.0 + r)
    w1 = r / (1.0 + r)
    w_ref[...] = jnp.concatenate([w0, w1], axis=1)

    ids = jnp.concatenate([a0, a1], axis=0)
    onehot = (ids == lax.broadcasted_iota(jnp.int32, (NT, E), 1)
              ).astype(jnp.float32)
    counts = jnp.sum(onehot, axis=0, keepdims=True)
    caps = jnp.ceil(counts / BM) * BM
    iu0 = lax.broadcasted_iota(jnp.int32, (E, E), 0)
    iu1 = lax.broadcasted_iota(jnp.int32, (E, E), 1)
    u_strict = (iu0 < iu1).astype(jnp.float32)
    u_incl = (iu0 <= iu1).astype(jnp.float32)
    offs = jnp.dot(caps, u_strict, precision=_HI)
    cumcaps = jnp.dot(caps, u_incl, precision=_HI)

    bio = (lax.broadcasted_iota(jnp.int32, (NB, E), 0) * BM).astype(jnp.float32)
    eob = jnp.sum((bio >= cumcaps).astype(jnp.int32), axis=1, keepdims=True)
    eob_ref[...] = jnp.minimum(eob, E - 1)
    act_ref[...] = (bio[:, 0:1] < cumcaps[:, E - 1:E]).astype(jnp.int32)

    CH = 512
    NCH = NT // CH
    pch = ((lax.broadcasted_iota(jnp.int32, (NCH, NT), 1) // CH)
           == lax.broadcasted_iota(jnp.int32, (NCH, NT), 0)
           ).astype(jnp.float32)
    s_all = jnp.dot(pch, onehot)
    l_tri = (lax.broadcasted_iota(jnp.int32, (NCH, NCH), 1)
             < lax.broadcasted_iota(jnp.int32, (NCH, NCH), 0)
             ).astype(jnp.float32)
    s_pre = jnp.dot(l_tri, s_all, precision=_HI)
    tri = (lax.broadcasted_iota(jnp.int32, (CH, CH), 1)
           < lax.broadcasted_iota(jnp.int32, (CH, CH), 0)
           ).astype(jnp.float32)
    for c in range(NCH):
        ohc = onehot[c * CH:(c + 1) * CH]
        rloc = jnp.dot(tri, ohc)
        base = s_pre[c:c + 1, :] + offs
        pos_c = jnp.sum((rloc + base) * ohc, axis=1, keepdims=True)
        pos_ref[pl.ds(c * CH, CH), :] = pos_c.astype(jnp.int32)


def _dispatch_body(act_ref, pos_ref, x_ref, xs_ref):
    m = pl.program_id(0)

    @pl.when(act_ref[m] == 1)
    def _():
        s0 = m * BM
        pos0 = pos_ref[0:1, :]
        pos1 = pos_ref[1:2, :]
        srow = lax.broadcasted_iota(jnp.int32, (BM, T), 0) + s0
        dmask = ((pos0 == srow) | (pos1 == srow)).astype(jnp.float32)
        xs_ref[...] = jnp.dot(dmask, x_ref[...])

    @pl.when(act_ref[m] == 0)
    def _z():
        xs_ref[...] = jnp.zeros((BM, C), jnp.float32)


def _ffn_body(eob_ref, act_ref, xs_ref, w1_ref, b1_ref, w2_ref, b2_ref,
              out_ref, acc_ref):
    h = pl.program_id(0)
    m = pl.program_id(1)

    @pl.when(act_ref[m] == 1)
    def _compute():
        hc = lax.dot_general(xs_ref[...], w1_ref[0],
                             (((1,), (1,)), ((), ()))) + b1_ref[0]
        hc = 0.5 * hc * (1.0 + lax.erf(hc * 0.7071067811865476))
        part = lax.dot_general(hc, w2_ref[0], (((1,), (1,)), ((), ())))

        @pl.when(h == 0)
        def _init():
            acc_ref[pl.ds(m * BM, BM), :] = part + b2_ref[0]

        @pl.when(h != 0)
        def _acc():
            acc_ref[pl.ds(m * BM, BM), :] += part

        @pl.when(h == NH - 1)
        def _flush():
            out_ref[...] = acc_ref[pl.ds(m * BM, BM), :]

    @pl.when((act_ref[m] == 0) & (h == NH - 1))
    def _zero():
        out_ref[...] = jnp.zeros((BM, C), jnp.float32)


def _combine_body(posT_ref, w_ref, outs_ref, res_ref):
    s = pl.program_id(1)
    p0 = posT_ref[:, 0:1]
    p1 = posT_ref[:, 1:2]
    w0 = w_ref[:, 0:1]
    w1 = w_ref[:, 1:2]
    scol = (lax.broadcasted_iota(jnp.int32, (p0.shape[0], BS), 1)
            + s * BS)
    cmask = ((scol == p0).astype(jnp.float32) * w0 +
             (scol == p1).astype(jnp.float32) * w1)
    part = jnp.dot(cmask, outs_ref[...])

    @pl.when(s == 0)
    def _init():
        res_ref[...] = part

    @pl.when(s != 0)
    def _acc():
        res_ref[...] += part


@jax.jit
def kernel(x, Wg, W1, b1, W2, b2):
    x_flat = x.reshape(T, C)
    pos, w, eob, act = pl.pallas_call(
        _router_body,
        out_shape=[
            jax.ShapeDtypeStruct((NT, 1), jnp.int32),
            jax.ShapeDtypeStruct((T, 2), jnp.float32),
            jax.ShapeDtypeStruct((NB, 1), jnp.int32),
            jax.ShapeDtypeStruct((NB, 1), jnp.int32),
        ],
        compiler_params=pltpu.CompilerParams(
            vmem_limit_bytes=128 * 1024 * 1024),
    )(x_flat, Wg)

    pos2 = pos.reshape(2, T)
    posT = pos2.transpose(1, 0)
    eob1 = eob.reshape(NB)
    act1 = act.reshape(NB)

    xs = pl.pallas_call(
        _dispatch_body,
        grid_spec=pltpu.PrefetchScalarGridSpec(
            num_scalar_prefetch=1,
            grid=(NB,),
            in_specs=[
                pl.BlockSpec((2, T), lambda m, act: (0, 0)),
                pl.BlockSpec((T, C), lambda m, act: (0, 0)),
            ],
            out_specs=pl.BlockSpec((BM, C), lambda m, act: (m, 0)),
        ),
        out_shape=jax.ShapeDtypeStruct((NBUF, C), jnp.float32),
        compiler_params=pltpu.CompilerParams(
            vmem_limit_bytes=128 * 1024 * 1024),
    )(act1, pos2, x_flat)

    outs = pl.pallas_call(
        _ffn_body,
        grid_spec=pltpu.PrefetchScalarGridSpec(
            num_scalar_prefetch=2,
            grid=(NH, NB),
            in_specs=[
                pl.BlockSpec((BM, C), lambda h, m, eob, act: (m, 0)),
                pl.BlockSpec((1, HC, C),
                             lambda h, m, eob, act: (eob[m], h, 0)),
                pl.BlockSpec((1, 1, HC),
                             lambda h, m, eob, act: (eob[m] * NH + h, 0, 0)),
                pl.BlockSpec((1, C, HC),
                             lambda h, m, eob, act: (eob[m], 0, h)),
                pl.BlockSpec((1, 1, C),
                             lambda h, m, eob, act: (eob[m], 0, 0)),
            ],
            out_specs=pl.BlockSpec(
                (BM, C),
                lambda h, m, eob, act: (jnp.where(h == NH - 1, m, NB), 0)),
            scratch_shapes=[pltpu.VMEM((NBUF, C), jnp.float32)],
        ),
        out_shape=jax.ShapeDtypeStruct((NBUF + BM, C), jnp.float32),
        compiler_params=pltpu.CompilerParams(
            vmem_limit_bytes=128 * 1024 * 1024),
    )(eob1, act1, xs, W1, b1.reshape(E * NH, 1, HC), W2,
      b2.reshape(E, 1, C))

    res = pl.pallas_call(
        _combine_body,
        grid=(T // BT, NBUF // BS),
        in_specs=[
            pl.BlockSpec((BT, 2), lambda i, s: (i, 0)),
            pl.BlockSpec((BT, 2), lambda i, s: (i, 0)),
            pl.BlockSpec((BS, C), lambda i, s: (s, 0)),
        ],
        out_specs=pl.BlockSpec((BT, C), lambda i, s: (i, 0)),
        out_shape=jax.ShapeDtypeStruct((T, C), jnp.float32),
        compiler_params=pltpu.CompilerParams(
            vmem_limit_bytes=128 * 1024 * 1024),
    )(posT, w, outs)
    return res.reshape(1, T, C)


# speedup vs baseline: 1.6824x; 1.0169x over previous
import functools

import jax
import jax.numpy as jnp
from jax import lax
from jax.experimental import pallas as pl
from jax.experimental.pallas import tpu as pltpu
from jax.experimental.pallas import tpu_sc as plsc

T = 2048
C = 1024
E = 8
HID = 4096
NT = T * 2
BM = 256
NBUF = NT + E * BM
NB = NBUF // BM
NH = 4
HC = HID // NH
BT = 1024
BS = 2048
_HI = jax.lax.Precision.HIGHEST


def _router_body(x_ref, wg_ref, pos_ref, w_ref, eob_ref, act_ref):
    x = x_ref[...]
    wg = wg_ref[...]
    logits = lax.dot_general(x, wg, (((1,), (1,)), ((), ())))
    logits = jnp.clip(logits, -20.0, 20.0)
    e_iota = lax.broadcasted_iota(jnp.int32, (T, E), 1)
    m0 = jnp.max(logits, axis=1, keepdims=True)
    a0 = jnp.min(jnp.where(logits == m0, e_iota, E), axis=1, keepdims=True)
    masked = jnp.where(e_iota == a0, -jnp.inf, logits)
    m1 = jnp.max(masked, axis=1, keepdims=True)
    a1 = jnp.min(jnp.where(masked == m1, e_iota, E), axis=1, keepdims=True)
    r = jnp.exp(m1 - m0)
    w0 = 1.0 / (1.0 + r)
    w1 = r / (1.0 + r)
    w_ref[...] = jnp.concatenate([w0, w1], axis=1)

    ids = jnp.concatenate([a0, a1], axis=0)
    onehot = (ids == lax.broadcasted_iota(jnp.int32, (NT, E), 1)
              ).astype(jnp.float32)
    counts = jnp.sum(onehot, axis=0, keepdims=True)
    caps = jnp.ceil(counts / BM) * BM
    iu0 = lax.broadcasted_iota(jnp.int32, (E, E), 0)
    iu1 = lax.broadcasted_iota(jnp.int32, (E, E), 1)
    u_strict = (iu0 < iu1).astype(jnp.float32)
    u_incl = (iu0 <= iu1).astype(jnp.float32)
    offs = jnp.dot(caps, u_strict, precision=_HI)
    cumcaps = jnp.dot(caps, u_incl, precision=_HI)

    bio = (lax.broadcasted_iota(jnp.int32, (NB, E), 0) * BM).astype(jnp.float32)
    eob = jnp.sum((bio >= cumcaps).astype(jnp.int32), axis=1, keepdims=True)
    eob_ref[...] = jnp.minimum(eob, E - 1)
    act_ref[...] = (bio[:, 0:1] < cumcaps[:, E - 1:E]).astype(jnp.int32)

    CH = 512
    NCH = NT // CH
    pch = ((lax.broadcasted_iota(jnp.int32, (NCH, NT), 1) // CH)
           == lax.broadcasted_iota(jnp.int32, (NCH, NT), 0)
           ).astype(jnp.float32)
    s_all = jnp.dot(pch, onehot)
    l_tri = (lax.broadcasted_iota(jnp.int32, (NCH, NCH), 1)
             < lax.broadcasted_iota(jnp.int32, (NCH, NCH), 0)
             ).astype(jnp.float32)
    s_pre = jnp.dot(l_tri, s_all, precision=_HI)
    tri = (lax.broadcasted_iota(jnp.int32, (CH, CH), 1)
           < lax.broadcasted_iota(jnp.int32, (CH, CH), 0)
           ).astype(jnp.float32)
    for c in range(NCH):
        ohc = onehot[c * CH:(c + 1) * CH]
        rloc = jnp.dot(tri, ohc)
        base = s_pre[c:c + 1, :] + offs
        pos_c = jnp.sum((rloc + base) * ohc, axis=1, keepdims=True)
        pos_ref[pl.ds(c * CH, CH), :] = pos_c.astype(jnp.int32)


def _dispatch_body(act_ref, pos_ref, x_ref, xs_ref):
    m = pl.program_id(0)

    @pl.when(act_ref[m] == 1)
    def _():
        s0 = m * BM
        pos0 = pos_ref[0:1, :]
        pos1 = pos_ref[1:2, :]
        srow = lax.broadcasted_iota(jnp.int32, (BM, T), 0) + s0
        dmask = ((pos0 == srow) | (pos1 == srow)).astype(jnp.float32)
        xs_ref[...] = jnp.dot(dmask, x_ref[...])

    @pl.when(act_ref[m] == 0)
    def _z():
        xs_ref[...] = jnp.zeros((BM, C), jnp.float32)


def _ffn_body(eob_ref, act_ref, xs_ref, w1_ref, b1_ref, w2_ref, b2_ref,
              out_ref, acc_ref):
    h = pl.program_id(0)
    m = pl.program_id(1)

    @pl.when(act_ref[m] == 1)
    def _compute():
        hc = lax.dot_general(xs_ref[...], w1_ref[0],
                             (((1,), (1,)), ((), ()))) + b1_ref[0]
        hc = 0.5 * hc * (1.0 + lax.erf(hc * 0.7071067811865476))
        part = lax.dot_general(hc, w2_ref[0], (((1,), (1,)), ((), ())))

        @pl.when(h == 0)
        def _init():
            acc_ref[pl.ds(m * BM, BM), :] = part + b2_ref[0]

        @pl.when(h != 0)
        def _acc():
            acc_ref[pl.ds(m * BM, BM), :] += part

        @pl.when(h == NH - 1)
        def _flush():
            out_ref[...] = acc_ref[pl.ds(m * BM, BM), :]

    @pl.when((act_ref[m] == 0) & (h == NH - 1))
    def _zero():
        out_ref[...] = jnp.zeros((BM, C), jnp.float32)


def _combine_body(posT_ref, w_ref, outs_ref, res_ref):
    s = pl.program_id(1)
    p0 = posT_ref[:, 0:1]
    p1 = posT_ref[:, 1:2]
    w0 = w_ref[:, 0:1]
    w1 = w_ref[:, 1:2]
    scol = (lax.broadcasted_iota(jnp.int32, (p0.shape[0], BS), 1)
            + s * BS)
    cmask = ((scol == p0).astype(jnp.float32) * w0 +
             (scol == p1).astype(jnp.float32) * w1)
    part = jnp.dot(cmask, outs_ref[...])

    @pl.when(s == 0)
    def _init():
        res_ref[...] = part

    @pl.when(s != 0)
    def _acc():
        res_ref[...] += part


@jax.jit
def kernel(x, Wg, W1, b1, W2, b2):
    x_flat = x.reshape(T, C)
    pos, w, eob, act = pl.pallas_call(
        _router_body,
        out_shape=[
            jax.ShapeDtypeStruct((NT, 1), jnp.int32),
            jax.ShapeDtypeStruct((T, 2), jnp.float32),
            jax.ShapeDtypeStruct((NB, 1), jnp.int32),
            jax.ShapeDtypeStruct((NB, 1), jnp.int32),
        ],
        compiler_params=pltpu.CompilerParams(
            vmem_limit_bytes=128 * 1024 * 1024),
    )(x_flat, Wg)

    pos2 = pos.reshape(2, T)
    posT = pos2.transpose(1, 0)
    eob1 = eob.reshape(NB)
    act1 = act.reshape(NB)

    xs = pl.pallas_call(
        _dispatch_body,
        grid_spec=pltpu.PrefetchScalarGridSpec(
            num_scalar_prefetch=1,
            grid=(NB,),
            in_specs=[
                pl.BlockSpec((2, T), lambda m, act: (0, 0)),
                pl.BlockSpec((T, C), lambda m, act: (0, 0)),
            ],
            out_specs=pl.BlockSpec((BM, C), lambda m, act: (m, 0)),
        ),
        out_shape=jax.ShapeDtypeStruct((NBUF, C), jnp.float32),
        compiler_params=pltpu.CompilerParams(
            vmem_limit_bytes=128 * 1024 * 1024),
    )(act1, pos2, x_flat)

    outs = pl.pallas_call(
        _ffn_body,
        grid_spec=pltpu.PrefetchScalarGridSpec(
            num_scalar_prefetch=2,
            grid=(NH, NB),
            in_specs=[
                pl.BlockSpec((BM, C), lambda h, m, eob, act: (m, 0)),
                pl.BlockSpec((1, HC, C),
                             lambda h, m, eob, act: (eob[m], h, 0)),
                pl.BlockSpec((1, 1, HC),
                             lambda h, m, eob, act: (eob[m] * NH + h, 0, 0)),
                pl.BlockSpec((1, C, HC),
                             lambda h, m, eob, act: (eob[m], 0, h)),
                pl.BlockSpec((1, 1, C),
                             lambda h, m, eob, act: (eob[m], 0, 0)),
            ],
            out_specs=pl.BlockSpec(
                (BM, C),
                lambda h, m, eob, act: (jnp.where(h == NH - 1, m, NB), 0)),
            scratch_shapes=[pltpu.VMEM((NBUF, C), jnp.float32)],
        ),
        out_shape=jax.ShapeDtypeStruct((NBUF + BM, C), jnp.float32),
        compiler_params=pltpu.CompilerParams(
            vmem_limit_bytes=128 * 1024 * 1024),
    )(eob1, act1, xs, W1, b1.reshape(E * NH, 1, HC), W2,
      b2.reshape(E, 1, C))

    res = pl.pallas_call(
        _combine_body,
        grid=(T // BT, NBUF // BS),
        in_specs=[
            pl.BlockSpec((BT, 2), lambda i, s: (i, 0)),
            pl.BlockSpec((BT, 2), lambda i, s: (i, 0)),
            pl.BlockSpec((BS, C), lambda i, s: (s, 0)),
        ],
        out_specs=pl.BlockSpec((BT, C), lambda i, s: (i, 0)),
        out_shape=jax.ShapeDtypeStruct((T, C), jnp.float32),
        compiler_params=pltpu.CompilerParams(
            vmem_limit_bytes=128 * 1024 * 1024),
    )(posT, w, outs)
    return res.reshape(1, T, C)
